# Initial kernel scaffold; baseline (speedup 1.0000x reference)
import functools

import jax
import jax.numpy as jnp
from jax import lax
from jax.experimental import pallas as pl
from jax.experimental.pallas import tpu as pltpu
from jax.experimental.pallas import tpu_sc as plsc

N_USERS = 20000
N_ITEMS = 30000
NN = N_USERS + N_ITEMS
NPAD = 50048
E = 800000
DH = 32
BATCH = 16384
NC, NS = 2, 16
EBLK = E // 128
NBLK = NPAD // 128
BPT = BATCH // NS

_MAGIC = jnp.int32(0x5F3759DF)


def _vrsqrt(x):
    i = plsc.bitcast(x, jnp.int32)
    y = plsc.bitcast(_MAGIC - (i >> 1), jnp.float32)
    for _ in range(3):
        y = y * (1.5 - 0.5 * x * y * y)
    return y


def _body(users, items, e0s, esrc, edst,
          partials, ubuf, carr, direp, dsrep,
          acc_sh, deg_sh,
          sidx, didx, rows, zrow, z1, ones1, dchunk, dib, dsb,
          e0c, stg, rb1, rb2, cch, rpc, pch, lu, li, gout, sem):
    c = lax.axis_index("c")
    s = lax.axis_index("s")
    coff = c * NPAD
    eb0 = (EBLK * s) // NS
    eb1 = (EBLK * (s + 1)) // NS
    nb0 = (NBLK * s) // NS
    nb1 = (NBLK * (s + 1)) // NS
    iota16 = lax.iota(jnp.int32, 16)
    zv = jnp.zeros((16,), jnp.float32)
    ov = jnp.ones((16,), jnp.float32)

    for g in range(8):
        z1[pl.ds(g * 16, 16)] = zv
        ones1[pl.ds(g * 16, 16)] = ov
    for r in range(128):
        zrow[r, pl.ds(0, 16)] = zv
        zrow[r, pl.ds(16, 16)] = zv

    def z_blk(i, carry):
        pltpu.sync_copy(zrow, acc_sh.at[pl.ds(i * 128, 128), :])
        pltpu.sync_copy(z1, deg_sh.at[pl.ds(i * 128, 128)])
        return carry
    lax.fori_loop(nb0, nb1, z_blk, 0)
    plsc.subcore_barrier()

    def deg_blk(i, carry):
        pltpu.sync_copy(edst.at[pl.ds(i * 128, 128)], didx.at[0])
        pltpu.sync_copy(ones1, deg_sh.at[didx.at[0]], add=True)
        return carry
    lax.fori_loop(eb0, eb1, deg_blk, 0)
    plsc.subcore_barrier()

    def prep_blk(i, carry):
        base = i * 128
        pltpu.sync_copy(deg_sh.at[pl.ds(base, 128)], dchunk)
        for g in range(8):
            dv = dchunk[pl.ds(g * 16, 16)] + 1.0
            dib[pl.ds(g * 16, 16)] = 1.0 / dv
            dsb[pl.ds(g * 16, 16)] = _vrsqrt(dv)
        pltpu.sync_copy(e0s.at[pl.ds(coff + base, 128), :], e0c)

        def rowfn(r, carry2):
            iv = jnp.full((16,), r, jnp.int32)
            divv = plsc.load_gather(dib, [iv])
            dsv = plsc.load_gather(dsb, [iv])
            for h in (0, 16):
                ev = e0c[r, pl.ds(h, 16)]
                stg[r, pl.ds(h, 16)] = dsv * ev
                rb1[r, pl.ds(h, 16)] = divv
                rb2[r, pl.ds(h, 16)] = dsv
            return carry2
        lax.fori_loop(0, 128, rowfn, 0)
        pltpu.sync_copy(stg, ubuf.at[pl.ds(coff + base, 128), :])
        pltpu.sync_copy(rb1, direp.at[pl.ds(coff + base, 128), :])
        pltpu.sync_copy(rb2, dsrep.at[pl.ds(coff + base, 128), :])
        return carry
    lax.fori_loop(nb0, nb1, prep_blk, 0)
    plsc.subcore_barrier()

    def layer_scatter():
        def eblk(i, carry):
            pltpu.sync_copy(esrc.at[pl.ds(i * 128, 128)], sidx.at[0])
            pltpu.sync_copy(edst.at[pl.ds(i * 128, 128)], didx.at[0])
            for g in range(8):
                sidx[0, pl.ds(g * 16, 16)] = sidx[0, pl.ds(g * 16, 16)] + coff
            pltpu.async_copy(ubuf.at[sidx.at[0]], rows.at[0], sem).wait()
            pltpu.sync_copy(rows.at[0], acc_sh.at[didx.at[0]], add=True)
            return carry
        lax.fori_loop(eb0, eb1, eblk, 0)

    def phase_c(k):
        def nblkfn(i, carry):
            base = i * 128
            pltpu.sync_copy(acc_sh.at[pl.ds(base, 128), :], cch)
            if k == 1:
                pltpu.sync_copy(direp.at[pl.ds(coff + base, 128), :], rpc)
                pltpu.sync_copy(cch, carr.at[pl.ds(coff + base, 128), :])
            elif k == 2:
                pltpu.sync_copy(direp.at[pl.ds(coff + base, 128), :], rpc)
                pltpu.sync_copy(carr.at[pl.ds(coff + base, 128), :], pch)
            else:
                pltpu.sync_copy(dsrep.at[pl.ds(coff + base, 128), :], rpc)
                pltpu.sync_copy(e0s.at[pl.ds(coff + base, 128), :], pch)

            def rowfn(r, carry2):
                for h in (0, 16):
                    cvv = cch[r, pl.ds(h, 16)]
                    rv = rpc[r, pl.ds(h, 16)]
                    if k == 1:
                        stg[r, pl.ds(h, 16)] = cvv * rv
                    elif k == 2:
                        stg[r, pl.ds(h, 16)] = (cvv - pch[r, pl.ds(h, 16)]) * rv
                    else:
                        stg[r, pl.ds(h, 16)] = (pch[r, pl.ds(h, 16)] + cvv * rv) * 0.25
                return carry2
            lax.fori_loop(0, 128, rowfn, 0)
            pltpu.sync_copy(stg, ubuf.at[pl.ds(coff + base, 128), :])
            return carry
        lax.fori_loop(nb0, nb1, nblkfn, 0)

    for k in (1, 2, 3):
        layer_scatter()
        plsc.subcore_barrier()
        phase_c(k)
        plsc.subcore_barrier()

    def bchunk(j, carry):
        boff = s * BPT + j * 128
        pltpu.sync_copy(users.at[pl.ds(boff, 128)], sidx.at[0])
        pltpu.sync_copy(items.at[pl.ds(boff, 128)], didx.at[0])
        for g in range(8):
            sidx[0, pl.ds(g * 16, 16)] = sidx[0, pl.ds(g * 16, 16)] + coff
            didx[0, pl.ds(g * 16, 16)] = didx[0, pl.ds(g * 16, 16)] + (coff + N_USERS)
        pltpu.async_copy(ubuf.at[sidx.at[0]], lu, sem).wait()
        pltpu.async_copy(ubuf.at[didx.at[0]], li, sem).wait()
        for g in range(8):
            riv = g * 16 + iota16
            acc = jnp.zeros((16,), jnp.float32)
            for col in range(32):
                cv = jnp.full((16,), col, jnp.int32)
                acc = acc + plsc.load_gather(lu, [riv, cv]) * plsc.load_gather(li, [riv, cv])
            gout[pl.ds(j * 128 + g * 16, 16)] = acc
        return carry
    lax.fori_loop(0, 8, bchunk, 0)
    pltpu.sync_copy(gout, partials.at[pl.ds(c * BATCH + s * BPT, BPT)])


_mesh = plsc.VectorSubcoreMesh(core_axis_name="c", subcore_axis_name="s",
                               num_cores=NC, num_subcores=NS)

_f32 = jnp.float32
_sc_call = functools.partial(
    pl.kernel,
    out_type=(
        jax.ShapeDtypeStruct((NC * BATCH,), _f32),
        jax.ShapeDtypeStruct((NC * NPAD, DH), _f32),
        jax.ShapeDtypeStruct((NC * NPAD, DH), _f32),
        jax.ShapeDtypeStruct((NC * NPAD, DH), _f32),
        jax.ShapeDtypeStruct((NC * NPAD, DH), _f32),
    ),
    mesh=_mesh,
    scratch_types=[
        pltpu.VMEM_SHARED((NPAD, DH), _f32),
        pltpu.VMEM_SHARED((NPAD,), _f32),
        pltpu.VMEM((2, 128), jnp.int32),
        pltpu.VMEM((2, 128), jnp.int32),
        pltpu.VMEM((2, 128, DH), _f32),
        pltpu.VMEM((128, DH), _f32),
        pltpu.VMEM((128,), _f32),
        pltpu.VMEM((128,), _f32),
        pltpu.VMEM((128,), _f32),
        pltpu.VMEM((128,), _f32),
        pltpu.VMEM((128,), _f32),
        pltpu.VMEM((128, DH), _f32),
        pltpu.VMEM((128, DH), _f32),
        pltpu.VMEM((128, DH), _f32),
        pltpu.VMEM((128, DH), _f32),
        pltpu.VMEM((128, DH), _f32),
        pltpu.VMEM((128, DH), _f32),
        pltpu.VMEM((128, DH), _f32),
        pltpu.VMEM((128, DH), _f32),
        pltpu.VMEM((128, DH), _f32),
        pltpu.VMEM((BPT,), _f32),
        pltpu.SemaphoreType.DMA,
    ],
)(_body)


def kernel(users, items, user_emb, item_emb, edge_src, edge_dst):
    all_emb = jnp.concatenate([user_emb, item_emb], axis=0)
    e0p = jnp.pad(all_emb, ((0, NPAD - NN), (0, 0)))
    e0s = e0p.reshape(NPAD, NC, DH).transpose(1, 0, 2).reshape(NC * NPAD, DH)
    partials = _sc_call(users, items, e0s, edge_src, edge_dst)[0]
    return partials[:BATCH] + partials[BATCH:]


# speedup vs baseline: 7.4793x; 7.4793x over previous
import functools

import jax
import jax.numpy as jnp
from jax import lax
from jax.experimental import pallas as pl
from jax.experimental.pallas import tpu as pltpu
from jax.experimental.pallas import tpu_sc as plsc

N_USERS = 20000
N_ITEMS = 30000
NN = N_USERS + N_ITEMS
NPAD = 50048
E = 800000
DH = 32
BATCH = 16384
NC, NS = 2, 16
EBLK = E // 128
RB = 64
RBLK = NPAD // RB
DBLK = NPAD // 128
BPT = BATCH // NS

_MAGIC = 0x5F3759DF


def _vrsqrt(x):
    i = lax.bitcast_convert_type(x, jnp.int32)
    y = lax.bitcast_convert_type(jnp.int32(_MAGIC) - (i >> 1), jnp.float32)
    for _ in range(3):
        y = y * (1.5 - 0.5 * x * y * y)
    return y


def _body(users, items, e0s, esrc, edst,
          partials, ubuf, carr, direp, dsrep,
          acc_sh, deg_sh,
          sidx, didx, rows, z1, ones1, dchunk, dib, dsb,
          stg, aux, cch, rpc, lu, li, gout, sem):
    c = lax.axis_index("c")
    s = lax.axis_index("s")
    coff = c * NPAD
    eb0 = (EBLK * s) // NS
    eb1 = (EBLK * (s + 1)) // NS
    rb0 = (RBLK * s) // NS
    rb1 = (RBLK * (s + 1)) // NS
    db0 = (DBLK * s) // NS
    db1 = (DBLK * (s + 1)) // NS
    iota16 = lax.iota(jnp.int32, 16)
    zv = jnp.zeros((16,), jnp.float32)
    ov = jnp.ones((16,), jnp.float32)

    for g in range(8):
        z1[pl.ds(g * 16, 16)] = zv
        ones1[pl.ds(g * 16, 16)] = ov
    for r in range(RB):
        stg[r, pl.ds(0, 16)] = zv
        stg[r, pl.ds(16, 16)] = zv

    def za_blk(i, carry):
        pltpu.sync_copy(stg, acc_sh.at[pl.ds(i * RB, RB), :])
        return carry
    lax.fori_loop(rb0, rb1, za_blk, 0)

    def zd_blk(i, carry):
        pltpu.sync_copy(z1, deg_sh.at[pl.ds(i * 128, 128)])
        return carry
    lax.fori_loop(db0, db1, zd_blk, 0)
    plsc.subcore_barrier()

    def deg_blk(i, carry):
        pltpu.sync_copy(edst.at[pl.ds(i * 128, 128)], didx.at[0])
        pltpu.sync_copy(ones1, deg_sh.at[didx.at[0]], add=True)
        return carry
    lax.fori_loop(eb0, eb1, deg_blk, 0)
    plsc.subcore_barrier()

    def prep_blk(i, carry):
        base = i * RB
        pltpu.sync_copy(deg_sh.at[pl.ds(base, RB)], dchunk)
        for g in range(RB // 16):
            dv = dchunk[pl.ds(g * 16, 16)] + 1.0
            dib[pl.ds(g * 16, 16)] = 1.0 / dv
            dsb[pl.ds(g * 16, 16)] = _vrsqrt(dv)
        pltpu.sync_copy(e0s.at[pl.ds(coff + base, RB), :], aux)

        def rowfn(r, carry2):
            iv = jnp.full((16,), r, jnp.int32)
            divv = plsc.load_gather(dib, [iv])
            dsv = plsc.load_gather(dsb, [iv])
            for h in (0, 16):
                ev = aux[r, pl.ds(h, 16)]
                stg[r, pl.ds(h, 16)] = dsv * ev
                cch[r, pl.ds(h, 16)] = divv
                rpc[r, pl.ds(h, 16)] = dsv
            return carry2
        lax.fori_loop(0, RB, rowfn, 0)
        pltpu.sync_copy(stg, ubuf.at[pl.ds(coff + base, RB), :])
        pltpu.sync_copy(cch, direp.at[pl.ds(coff + base, RB), :])
        pltpu.sync_copy(rpc, dsrep.at[pl.ds(coff + base, RB), :])
        return carry
    lax.fori_loop(rb0, rb1, prep_blk, 0)
    plsc.subcore_barrier()

    def layer_scatter():
        def eblk(i, carry):
            pltpu.sync_copy(esrc.at[pl.ds(i * 128, 128)], sidx.at[0])
            pltpu.sync_copy(edst.at[pl.ds(i * 128, 128)], didx.at[0])
            for g in range(8):
                sidx[0, pl.ds(g * 16, 16)] = sidx[0, pl.ds(g * 16, 16)] + coff
            pltpu.async_copy(ubuf.at[sidx.at[0]], rows.at[0], sem).wait()
            pltpu.sync_copy(rows.at[0], acc_sh.at[didx.at[0]], add=True)
            return carry
        lax.fori_loop(eb0, eb1, eblk, 0)

    def phase_c(k):
        def nblkfn(i, carry):
            base = i * RB
            pltpu.sync_copy(acc_sh.at[pl.ds(base, RB), :], cch)
            if k == 1:
                pltpu.sync_copy(direp.at[pl.ds(coff + base, RB), :], rpc)
                pltpu.sync_copy(cch, carr.at[pl.ds(coff + base, RB), :])
            elif k == 2:
                pltpu.sync_copy(direp.at[pl.ds(coff + base, RB), :], rpc)
                pltpu.sync_copy(carr.at[pl.ds(coff + base, RB), :], aux)
            else:
                pltpu.sync_copy(dsrep.at[pl.ds(coff + base, RB), :], rpc)
                pltpu.sync_copy(e0s.at[pl.ds(coff + base, RB), :], aux)

            def rowfn(r, carry2):
                for h in (0, 16):
                    cvv = cch[r, pl.ds(h, 16)]
                    rv = rpc[r, pl.ds(h, 16)]
                    if k == 1:
                        stg[r, pl.ds(h, 16)] = cvv * rv
                    elif k == 2:
                        stg[r, pl.ds(h, 16)] = (cvv - aux[r, pl.ds(h, 16)]) * rv
                    else:
                        stg[r, pl.ds(h, 16)] = (aux[r, pl.ds(h, 16)] + cvv * rv) * 0.25
                return carry2
            lax.fori_loop(0, RB, rowfn, 0)
            pltpu.sync_copy(stg, ubuf.at[pl.ds(coff + base, RB), :])
            return carry
        lax.fori_loop(rb0, rb1, nblkfn, 0)

    for k in (1, 2, 3):
        layer_scatter()
        plsc.subcore_barrier()
        phase_c(k)
        plsc.subcore_barrier()

    def bchunk(j, carry):
        boff = s * BPT + j * 128
        pltpu.sync_copy(users.at[pl.ds(boff, 128)], sidx.at[0])
        pltpu.sync_copy(items.at[pl.ds(boff, 128)], didx.at[0])
        for g in range(8):
            sidx[0, pl.ds(g * 16, 16)] = sidx[0, pl.ds(g * 16, 16)] + coff
            didx[0, pl.ds(g * 16, 16)] = didx[0, pl.ds(g * 16, 16)] + (coff + N_USERS)
        pltpu.async_copy(ubuf.at[sidx.at[0]], lu, sem).wait()
        pltpu.async_copy(ubuf.at[didx.at[0]], li, sem).wait()
        for g in range(8):
            riv = g * 16 + iota16
            acc = jnp.zeros((16,), jnp.float32)
            for col in range(32):
                cv = jnp.full((16,), col, jnp.int32)
                acc = acc + plsc.load_gather(lu, [riv, cv]) * plsc.load_gather(li, [riv, cv])
            gout[pl.ds(j * 128 + g * 16, 16)] = acc
        return carry
    lax.fori_loop(0, 8, bchunk, 0)
    pltpu.sync_copy(gout, partials.at[pl.ds(c * BATCH + s * BPT, BPT)])


_mesh = plsc.VectorSubcoreMesh(core_axis_name="c", subcore_axis_name="s",
                               num_cores=NC, num_subcores=NS)

_f32 = jnp.float32
_sc_call = functools.partial(
    pl.kernel,
    out_type=(
        jax.ShapeDtypeStruct((NC * BATCH,), _f32),
        jax.ShapeDtypeStruct((NC * NPAD, DH), _f32),
        jax.ShapeDtypeStruct((NC * NPAD, DH), _f32),
        jax.ShapeDtypeStruct((NC * NPAD, DH), _f32),
        jax.ShapeDtypeStruct((NC * NPAD, DH), _f32),
    ),
    mesh=_mesh,
    compiler_params=pltpu.CompilerParams(needs_layout_passes=False,
                                         use_tc_tiling_on_sc=False),
    scratch_types=[
        pltpu.VMEM_SHARED((NPAD, DH), _f32),
        pltpu.VMEM_SHARED((NPAD,), _f32),
        pltpu.VMEM((2, 128), jnp.int32),
        pltpu.VMEM((2, 128), jnp.int32),
        pltpu.VMEM((1, 128, DH), _f32),
        pltpu.VMEM((128,), _f32),
        pltpu.VMEM((128,), _f32),
        pltpu.VMEM((RB,), _f32),
        pltpu.VMEM((RB,), _f32),
        pltpu.VMEM((RB,), _f32),
        pltpu.VMEM((RB, DH), _f32),
        pltpu.VMEM((RB, DH), _f32),
        pltpu.VMEM((RB, DH), _f32),
        pltpu.VMEM((RB, DH), _f32),
        pltpu.VMEM((128, DH), _f32),
        pltpu.VMEM((128, DH), _f32),
        pltpu.VMEM((BPT,), _f32),
        pltpu.SemaphoreType.DMA,
    ],
)(_body)


def kernel(users, items, user_emb, item_emb, edge_src, edge_dst):
    all_emb = jnp.concatenate([user_emb, item_emb], axis=0)
    e0p = jnp.pad(all_emb, ((0, NPAD - NN), (0, 0)))
    e0s = e0p.reshape(NPAD, NC, DH).transpose(1, 0, 2).reshape(NC * NPAD, DH)
    partials = _sc_call(users, items, e0s, edge_src, edge_dst)[0]
    return partials[:BATCH] + partials[BATCH:]


# speedup vs baseline: 10.6115x; 1.4188x over previous
import functools

import jax
import jax.numpy as jnp
from jax import lax
from jax.experimental import pallas as pl
from jax.experimental.pallas import tpu as pltpu
from jax.experimental.pallas import tpu_sc as plsc

N_USERS = 20000
N_ITEMS = 30000
NN = N_USERS + N_ITEMS
NPAD = 50048
E = 800000
DH = 32
BATCH = 16384
NC, NS = 2, 16
EBLK = E // 128
RB = 64
RBLK = NPAD // RB
DBLK = NPAD // 128
BPT = BATCH // NS

_MAGIC = 0x5F3759DF


def _vrsqrt(x):
    i = lax.bitcast_convert_type(x, jnp.int32)
    y = lax.bitcast_convert_type(jnp.int32(_MAGIC) - (i >> 1), jnp.float32)
    for _ in range(3):
        y = y * (1.5 - 0.5 * x * y * y)
    return y


def _body(users, items, e0s, esrc, edst,
          partials, ubuf, carr, direp, dsrep,
          acc_sh, deg_sh,
          sidx, didx, rows, z1, ones1, dchunk, dib, dsb,
          stg, aux, cch, rpc, lu, li, gout, sem):
    c = lax.axis_index("c")
    s = lax.axis_index("s")
    coff = c * NPAD
    eb0 = (EBLK * s) // NS
    eb1 = (EBLK * (s + 1)) // NS
    rb0 = (RBLK * s) // NS
    rb1 = (RBLK * (s + 1)) // NS
    db0 = (DBLK * s) // NS
    db1 = (DBLK * (s + 1)) // NS
    iota16 = lax.iota(jnp.int32, 16)
    zv = jnp.zeros((16,), jnp.float32)
    ov = jnp.ones((16,), jnp.float32)

    for g in range(8):
        z1[pl.ds(g * 16, 16)] = zv
        ones1[pl.ds(g * 16, 16)] = ov
    for r in range(RB):
        stg[r, pl.ds(0, 16)] = zv
        stg[r, pl.ds(16, 16)] = zv

    def za_blk(i, carry):
        pltpu.sync_copy(stg, acc_sh.at[pl.ds(i * RB, RB), :])
        return carry
    lax.fori_loop(rb0, rb1, za_blk, 0)

    def zd_blk(i, carry):
        pltpu.sync_copy(z1, deg_sh.at[pl.ds(i * 128, 128)])
        return carry
    lax.fori_loop(db0, db1, zd_blk, 0)
    plsc.subcore_barrier()

    def deg_blk(i, carry):
        pltpu.sync_copy(edst.at[pl.ds(i * 128, 128)], didx.at[0])
        pltpu.sync_copy(ones1, deg_sh.at[didx.at[0]], add=True)
        return carry
    lax.fori_loop(eb0, eb1, deg_blk, 0)
    plsc.subcore_barrier()

    def prep_blk(i, carry):
        base = i * RB
        pltpu.sync_copy(deg_sh.at[pl.ds(base, RB)], dchunk)
        for g in range(RB // 16):
            dv = dchunk[pl.ds(g * 16, 16)] + 1.0
            dib[pl.ds(g * 16, 16)] = 1.0 / dv
            dsb[pl.ds(g * 16, 16)] = _vrsqrt(dv)
        pltpu.sync_copy(e0s.at[pl.ds(coff + base, RB), :], aux)

        def rowfn(r, carry2):
            iv = jnp.full((16,), r, jnp.int32)
            divv = plsc.load_gather(dib, [iv])
            dsv = plsc.load_gather(dsb, [iv])
            for h in (0, 16):
                ev = aux[r, pl.ds(h, 16)]
                stg[r, pl.ds(h, 16)] = dsv * ev
                cch[r, pl.ds(h, 16)] = divv
                rpc[r, pl.ds(h, 16)] = dsv
            return carry2
        lax.fori_loop(0, RB, rowfn, 0)
        pltpu.sync_copy(stg, ubuf.at[pl.ds(coff + base, RB), :])
        pltpu.sync_copy(cch, direp.at[pl.ds(coff + base, RB), :])
        pltpu.sync_copy(rpc, dsrep.at[pl.ds(coff + base, RB), :])
        return carry
    lax.fori_loop(rb0, rb1, prep_blk, 0)
    plsc.subcore_barrier()

    def load_blk(i, q):
        pltpu.sync_copy(esrc.at[pl.ds(i * 128, 128)], sidx.at[q])
        pltpu.sync_copy(edst.at[pl.ds(i * 128, 128)], didx.at[q])
        for g in range(8):
            sidx[q, pl.ds(g * 16, 16)] = sidx[q, pl.ds(g * 16, 16)] + coff
        pltpu.async_copy(ubuf.at[sidx.at[q]], rows.at[q], sem.at[q])

    def layer_scatter():
        load_blk(eb0, 0)

        def eblk(i, carry):
            p = (i - eb0) & 1

            @pl.when(i + 1 < eb1)
            def _prefetch():
                load_blk(i + 1, 1 - p)

            pltpu.make_async_copy(ubuf.at[sidx.at[p]], rows.at[p], sem.at[p]).wait()
            pltpu.sync_copy(rows.at[p], acc_sh.at[didx.at[p]], add=True)
            return carry
        lax.fori_loop(eb0, eb1, eblk, 0)

    def phase_c(k):
        def nblkfn(i, carry):
            base = i * RB
            pltpu.sync_copy(acc_sh.at[pl.ds(base, RB), :], cch)
            if k == 1:
                pltpu.sync_copy(direp.at[pl.ds(coff + base, RB), :], rpc)
                pltpu.sync_copy(cch, carr.at[pl.ds(coff + base, RB), :])
            elif k == 2:
                pltpu.sync_copy(direp.at[pl.ds(coff + base, RB), :], rpc)
                pltpu.sync_copy(carr.at[pl.ds(coff + base, RB), :], aux)
            else:
                pltpu.sync_copy(dsrep.at[pl.ds(coff + base, RB), :], rpc)
                pltpu.sync_copy(e0s.at[pl.ds(coff + base, RB), :], aux)

            def rowfn(r, carry2):
                for h in (0, 16):
                    cvv = cch[r, pl.ds(h, 16)]
                    rv = rpc[r, pl.ds(h, 16)]
                    if k == 1:
                        stg[r, pl.ds(h, 16)] = cvv * rv
                    elif k == 2:
                        stg[r, pl.ds(h, 16)] = (cvv - aux[r, pl.ds(h, 16)]) * rv
                    else:
                        stg[r, pl.ds(h, 16)] = (aux[r, pl.ds(h, 16)] + cvv * rv) * 0.25
                return carry2
            lax.fori_loop(0, RB, rowfn, 0)
            pltpu.sync_copy(stg, ubuf.at[pl.ds(coff + base, RB), :])
            return carry
        lax.fori_loop(rb0, rb1, nblkfn, 0)

    for k in (1, 2, 3):
        layer_scatter()
        plsc.subcore_barrier()
        phase_c(k)
        plsc.subcore_barrier()

    def bchunk(j, carry):
        boff = s * BPT + j * 128
        pltpu.sync_copy(users.at[pl.ds(boff, 128)], sidx.at[0])
        pltpu.sync_copy(items.at[pl.ds(boff, 128)], didx.at[0])
        for g in range(8):
            sidx[0, pl.ds(g * 16, 16)] = sidx[0, pl.ds(g * 16, 16)] + coff
            didx[0, pl.ds(g * 16, 16)] = didx[0, pl.ds(g * 16, 16)] + (coff + N_USERS)
        pltpu.async_copy(ubuf.at[sidx.at[0]], lu, sem.at[0]).wait()
        pltpu.async_copy(ubuf.at[didx.at[0]], li, sem.at[0]).wait()
        for g in range(8):
            riv = g * 16 + iota16
            acc = jnp.zeros((16,), jnp.float32)
            for col in range(32):
                cv = jnp.full((16,), col, jnp.int32)
                acc = acc + plsc.load_gather(lu, [riv, cv]) * plsc.load_gather(li, [riv, cv])
            gout[pl.ds(j * 128 + g * 16, 16)] = acc
        return carry
    lax.fori_loop(0, 8, bchunk, 0)
    pltpu.sync_copy(gout, partials.at[pl.ds(c * BATCH + s * BPT, BPT)])


_mesh = plsc.VectorSubcoreMesh(core_axis_name="c", subcore_axis_name="s",
                               num_cores=NC, num_subcores=NS)

_f32 = jnp.float32
_sc_call = functools.partial(
    pl.kernel,
    out_type=(
        jax.ShapeDtypeStruct((NC * BATCH,), _f32),
        jax.ShapeDtypeStruct((NC * NPAD, DH), _f32),
        jax.ShapeDtypeStruct((NC * NPAD, DH), _f32),
        jax.ShapeDtypeStruct((NC * NPAD, DH), _f32),
        jax.ShapeDtypeStruct((NC * NPAD, DH), _f32),
    ),
    mesh=_mesh,
    compiler_params=pltpu.CompilerParams(needs_layout_passes=False,
                                         use_tc_tiling_on_sc=False),
    scratch_types=[
        pltpu.VMEM_SHARED((NPAD, DH), _f32),
        pltpu.VMEM_SHARED((NPAD,), _f32),
        pltpu.VMEM((2, 128), jnp.int32),
        pltpu.VMEM((2, 128), jnp.int32),
        pltpu.VMEM((2, 128, DH), _f32),
        pltpu.VMEM((128,), _f32),
        pltpu.VMEM((128,), _f32),
        pltpu.VMEM((RB,), _f32),
        pltpu.VMEM((RB,), _f32),
        pltpu.VMEM((RB,), _f32),
        pltpu.VMEM((RB, DH), _f32),
        pltpu.VMEM((RB, DH), _f32),
        pltpu.VMEM((RB, DH), _f32),
        pltpu.VMEM((RB, DH), _f32),
        pltpu.VMEM((128, DH), _f32),
        pltpu.VMEM((128, DH), _f32),
        pltpu.VMEM((BPT,), _f32),
        pltpu.SemaphoreType.DMA((2,)),
    ],
)(_body)


def kernel(users, items, user_emb, item_emb, edge_src, edge_dst):
    all_emb = jnp.concatenate([user_emb, item_emb], axis=0)
    e0p = jnp.pad(all_emb, ((0, NPAD - NN), (0, 0)))
    e0s = e0p.reshape(NPAD, NC, DH).transpose(1, 0, 2).reshape(NC * NPAD, DH)
    partials = _sc_call(users, items, e0s, edge_src, edge_dst)[0]
    return partials[:BATCH] + partials[BATCH:]


# speedup vs baseline: 18.8042x; 1.7721x over previous
import functools

import jax
import jax.numpy as jnp
from jax import lax
from jax.experimental import pallas as pl
from jax.experimental.pallas import tpu as pltpu
from jax.experimental.pallas import tpu_sc as plsc

N_USERS = 20000
N_ITEMS = 30000
NN = N_USERS + N_ITEMS
NPAD = 50048
E = 800000
DH = 32
BATCH = 16384
NC, NS = 2, 16
EBLK = E // 128
RB = 64
RBLK = NPAD // RB
DBLK = NPAD // 128
BPT = BATCH // NS

_MAGIC = 0x5F3759DF


def _vrsqrt(x):
    i = lax.bitcast_convert_type(x, jnp.int32)
    y = lax.bitcast_convert_type(jnp.int32(_MAGIC) - (i >> 1), jnp.float32)
    for _ in range(3):
        y = y * (1.5 - 0.5 * x * y * y)
    return y


def _body(users, items, e0s, esrc, edst,
          partials, ubuf, carr, direp, dsrep,
          acc_sh, deg_sh,
          sidx, didx, rows, z1, ones1, dchunk, dib, dsb,
          stg, aux, cch, rpc, lu, li, gout, isem, gsem, ssem):
    c = lax.axis_index("c")
    s = lax.axis_index("s")
    coff = c * NPAD
    eb0 = (EBLK * s) // NS
    eb1 = (EBLK * (s + 1)) // NS
    rb0 = (RBLK * s) // NS
    rb1 = (RBLK * (s + 1)) // NS
    db0 = (DBLK * s) // NS
    db1 = (DBLK * (s + 1)) // NS
    iota16 = lax.iota(jnp.int32, 16)
    zv = jnp.zeros((16,), jnp.float32)
    ov = jnp.ones((16,), jnp.float32)

    for g in range(8):
        z1[pl.ds(g * 16, 16)] = zv
        ones1[pl.ds(g * 16, 16)] = ov
    for r in range(RB):
        stg[r, pl.ds(0, 16)] = zv
        stg[r, pl.ds(16, 16)] = zv

    def za_blk(i, carry):
        pltpu.sync_copy(stg, acc_sh.at[pl.ds(i * RB, RB), :])
        return carry
    lax.fori_loop(rb0, rb1, za_blk, 0)

    def zd_blk(i, carry):
        pltpu.sync_copy(z1, deg_sh.at[pl.ds(i * 128, 128)])
        return carry
    lax.fori_loop(db0, db1, zd_blk, 0)
    plsc.subcore_barrier()

    def dfire(i):
        pltpu.async_copy(edst.at[pl.ds(i * 128, 128)], didx.at[i & 3], isem.at[i & 3])

    def dwait(i):
        pltpu.make_async_copy(edst.at[pl.ds(i * 128, 128)], didx.at[i & 3],
                              isem.at[i & 3]).wait()

    def dscat_wait(i):
        pltpu.make_async_copy(ones1, deg_sh.at[didx.at[i & 3]], ssem.at[i & 1]).wait()

    dfire(eb0)

    @pl.when(eb0 + 1 < eb1)
    def _d1():
        dfire(eb0 + 1)

    def deg_blk(i, carry):
        @pl.when(i - 2 >= eb0)
        def _w():
            dscat_wait(i - 2)

        @pl.when(i + 2 < eb1)
        def _f():
            dfire(i + 2)
        dwait(i)
        pltpu.async_copy(ones1, deg_sh.at[didx.at[i & 3]], ssem.at[i & 1], add=True)
        return carry
    lax.fori_loop(eb0, eb1, deg_blk, 0)

    @pl.when(eb1 - 2 >= eb0)
    def _dw2():
        dscat_wait(eb1 - 2)
    dscat_wait(eb1 - 1)
    plsc.subcore_barrier()

    def prep_blk(i, carry):
        base = i * RB
        pltpu.sync_copy(deg_sh.at[pl.ds(base, RB)], dchunk)
        for g in range(RB // 16):
            dv = dchunk[pl.ds(g * 16, 16)] + 1.0
            dib[pl.ds(g * 16, 16)] = 1.0 / dv
            dsb[pl.ds(g * 16, 16)] = _vrsqrt(dv)
        pltpu.sync_copy(e0s.at[pl.ds(coff + base, RB), :], aux)

        def rowfn(r, carry2):
            iv = jnp.full((16,), r, jnp.int32)
            divv = plsc.load_gather(dib, [iv])
            dsv = plsc.load_gather(dsb, [iv])
            for h in (0, 16):
                ev = aux[r, pl.ds(h, 16)]
                stg[r, pl.ds(h, 16)] = dsv * ev
                cch[r, pl.ds(h, 16)] = divv
                rpc[r, pl.ds(h, 16)] = dsv
            return carry2
        lax.fori_loop(0, RB, rowfn, 0)
        pltpu.sync_copy(stg, ubuf.at[pl.ds(coff + base, RB), :])
        pltpu.sync_copy(cch, direp.at[pl.ds(coff + base, RB), :])
        pltpu.sync_copy(rpc, dsrep.at[pl.ds(coff + base, RB), :])
        return carry
    lax.fori_loop(rb0, rb1, prep_blk, 0)
    plsc.subcore_barrier()

    def idx_fire(i):
        q = i & 3
        pltpu.async_copy(esrc.at[pl.ds(i * 128, 128)], sidx.at[q], isem.at[q])
        pltpu.async_copy(edst.at[pl.ds(i * 128, 128)], didx.at[q], isem.at[q])

    def idx_wait(i):
        q = i & 3
        pltpu.make_async_copy(esrc.at[pl.ds(i * 128, 128)], sidx.at[q],
                              isem.at[q]).wait()
        pltpu.make_async_copy(edst.at[pl.ds(i * 128, 128)], didx.at[q],
                              isem.at[q]).wait()

    def gather_fire(i):
        q = i & 3
        for g in range(8):
            sidx[q, pl.ds(g * 16, 16)] = sidx[q, pl.ds(g * 16, 16)] + coff
        pltpu.async_copy(ubuf.at[sidx.at[q]], rows.at[i & 1], gsem.at[i & 1])

    def gather_wait(i):
        pltpu.make_async_copy(ubuf.at[sidx.at[i & 3]], rows.at[i & 1],
                              gsem.at[i & 1]).wait()

    def ascat_wait(i):
        pltpu.make_async_copy(rows.at[i & 1], acc_sh.at[didx.at[i & 3]],
                              ssem.at[i & 1]).wait()

    def layer_scatter():
        idx_fire(eb0)

        @pl.when(eb0 + 1 < eb1)
        def _i1():
            idx_fire(eb0 + 1)
        idx_wait(eb0)
        gather_fire(eb0)

        def eblk(i, carry):
            @pl.when(i - 1 >= eb0)
            def _sw():
                ascat_wait(i - 1)

            @pl.when(i + 2 < eb1)
            def _if():
                idx_fire(i + 2)

            @pl.when(i + 1 < eb1)
            def _gf():
                idx_wait(i + 1)
                gather_fire(i + 1)
            gather_wait(i)
            pltpu.async_copy(rows.at[i & 1], acc_sh.at[didx.at[i & 3]],
                             ssem.at[i & 1], add=True)
            return carry
        lax.fori_loop(eb0, eb1, eblk, 0)
        ascat_wait(eb1 - 1)

    def phase_c(k):
        def nblkfn(i, carry):
            base = i * RB
            pltpu.sync_copy(acc_sh.at[pl.ds(base, RB), :], cch)
            if k == 1:
                pltpu.sync_copy(direp.at[pl.ds(coff + base, RB), :], rpc)
                pltpu.sync_copy(cch, carr.at[pl.ds(coff + base, RB), :])
            elif k == 2:
                pltpu.sync_copy(direp.at[pl.ds(coff + base, RB), :], rpc)
                pltpu.sync_copy(carr.at[pl.ds(coff + base, RB), :], aux)
            else:
                pltpu.sync_copy(dsrep.at[pl.ds(coff + base, RB), :], rpc)
                pltpu.sync_copy(e0s.at[pl.ds(coff + base, RB), :], aux)

            def rowfn(r, carry2):
                for h in (0, 16):
                    cvv = cch[r, pl.ds(h, 16)]
                    rv = rpc[r, pl.ds(h, 16)]
                    if k == 1:
                        stg[r, pl.ds(h, 16)] = cvv * rv
                    elif k == 2:
                        stg[r, pl.ds(h, 16)] = (cvv - aux[r, pl.ds(h, 16)]) * rv
                    else:
                        stg[r, pl.ds(h, 16)] = (aux[r, pl.ds(h, 16)] + cvv * rv) * 0.25
                return carry2
            lax.fori_loop(0, RB, rowfn, 0)
            pltpu.sync_copy(stg, ubuf.at[pl.ds(coff + base, RB), :])
            return carry
        lax.fori_loop(rb0, rb1, nblkfn, 0)

    for k in (1, 2, 3):
        layer_scatter()
        plsc.subcore_barrier()
        phase_c(k)
        plsc.subcore_barrier()

    def bchunk(j, carry):
        boff = s * BPT + j * 128
        pltpu.sync_copy(users.at[pl.ds(boff, 128)], sidx.at[0])
        pltpu.sync_copy(items.at[pl.ds(boff, 128)], didx.at[0])
        for g in range(8):
            sidx[0, pl.ds(g * 16, 16)] = sidx[0, pl.ds(g * 16, 16)] + coff
            didx[0, pl.ds(g * 16, 16)] = didx[0, pl.ds(g * 16, 16)] + (coff + N_USERS)
        pltpu.async_copy(ubuf.at[sidx.at[0]], lu, gsem.at[0]).wait()
        pltpu.async_copy(ubuf.at[didx.at[0]], li, gsem.at[0]).wait()
        for g in range(8):
            riv = g * 16 + iota16
            acc = jnp.zeros((16,), jnp.float32)
            for col in range(32):
                cv = jnp.full((16,), col, jnp.int32)
                acc = acc + plsc.load_gather(lu, [riv, cv]) * plsc.load_gather(li, [riv, cv])
            gout[pl.ds(j * 128 + g * 16, 16)] = acc
        return carry
    lax.fori_loop(0, 8, bchunk, 0)
    pltpu.sync_copy(gout, partials.at[pl.ds(c * BATCH + s * BPT, BPT)])


_mesh = plsc.VectorSubcoreMesh(core_axis_name="c", subcore_axis_name="s",
                               num_cores=NC, num_subcores=NS)

_f32 = jnp.float32
_sc_call = functools.partial(
    pl.kernel,
    out_type=(
        jax.ShapeDtypeStruct((NC * BATCH,), _f32),
        jax.ShapeDtypeStruct((NC * NPAD, DH), _f32),
        jax.ShapeDtypeStruct((NC * NPAD, DH), _f32),
        jax.ShapeDtypeStruct((NC * NPAD, DH), _f32),
        jax.ShapeDtypeStruct((NC * NPAD, DH), _f32),
    ),
    mesh=_mesh,
    compiler_params=pltpu.CompilerParams(needs_layout_passes=False,
                                         use_tc_tiling_on_sc=False),
    scratch_types=[
        pltpu.VMEM_SHARED((NPAD, DH), _f32),
        pltpu.VMEM_SHARED((NPAD,), _f32),
        pltpu.VMEM((4, 128), jnp.int32),
        pltpu.VMEM((4, 128), jnp.int32),
        pltpu.VMEM((2, 128, DH), _f32),
        pltpu.VMEM((128,), _f32),
        pltpu.VMEM((128,), _f32),
        pltpu.VMEM((RB,), _f32),
        pltpu.VMEM((RB,), _f32),
        pltpu.VMEM((RB,), _f32),
        pltpu.VMEM((RB, DH), _f32),
        pltpu.VMEM((RB, DH), _f32),
        pltpu.VMEM((RB, DH), _f32),
        pltpu.VMEM((RB, DH), _f32),
        pltpu.VMEM((128, DH), _f32),
        pltpu.VMEM((128, DH), _f32),
        pltpu.VMEM((BPT,), _f32),
        pltpu.SemaphoreType.DMA((4,)),
        pltpu.SemaphoreType.DMA((2,)),
        pltpu.SemaphoreType.DMA((2,)),
    ],
)(_body)


def kernel(users, items, user_emb, item_emb, edge_src, edge_dst):
    all_emb = jnp.concatenate([user_emb, item_emb], axis=0)
    e0p = jnp.pad(all_emb, ((0, NPAD - NN), (0, 0)))
    e0s = e0p.reshape(NPAD, NC, DH).transpose(1, 0, 2).reshape(NC * NPAD, DH)
    partials = _sc_call(users, items, e0s, edge_src, edge_dst)[0]
    return partials[:BATCH] + partials[BATCH:]
